# Initial kernel scaffold; baseline (speedup 1.0000x reference)
import jax
import jax.numpy as jnp
from jax.experimental import pallas as pl


def kernel(x, edge_index, edge_type, rel_embedding):
    raise NotImplementedError("write your pallas kernel here")


# speedup vs baseline: 2.1511x; 2.1511x over previous
import functools

import jax
import jax.numpy as jnp
from jax import lax
from jax.experimental import pallas as pl
from jax.experimental.pallas import tpu as pltpu
from jax.experimental.pallas import tpu_sc as plsc

N_NODES = 10000
N_EDGES = 320000
D = 128
NUM_REL = 237

_INFO = plsc.get_sparse_core_info()
NC = _INFO.num_cores
NS = _INFO.num_subcores
NW = NC * NS
LANES = 16
VPR = D // LANES

E_PER_W = N_EDGES // NW
CHUNK = 80
N_ITER = E_PER_W // CHUNK


def _make_kernel():
    mesh = plsc.VectorSubcoreMesh(core_axis_name="c", subcore_axis_name="s")

    @functools.partial(
        pl.kernel,
        out_type=jax.ShapeDtypeStruct((N_EDGES,), jnp.float32),
        mesh=mesh,
        compiler_params=pltpu.CompilerParams(needs_layout_passes=False),
        scratch_types=[
            pltpu.VMEM((CHUNK,), jnp.int32),
            pltpu.VMEM((CHUNK,), jnp.int32),
            pltpu.VMEM((CHUNK,), jnp.int32),
            pltpu.VMEM((CHUNK, D), jnp.float32),
            pltpu.VMEM((CHUNK, D), jnp.float32),
            pltpu.VMEM((CHUNK, D), jnp.float32),
            pltpu.VMEM((CHUNK,), jnp.float32),
            pltpu.SemaphoreType.DMA,
        ],
    )
    def k(x_hbm, h_hbm, t_hbm, et_hbm, rel_hbm, out_hbm,
          hidx, tidx, etidx, bufh, buft, bufr, outbuf, sem):
        wid = lax.axis_index("s") * NC + lax.axis_index("c")
        wbase = wid * E_PER_W

        lane = lax.iota(jnp.int32, 16)

        def body(i, _):
            base = wbase + i * CHUNK
            pltpu.sync_copy(h_hbm.at[pl.ds(base, CHUNK)], hidx)
            pltpu.sync_copy(t_hbm.at[pl.ds(base, CHUNK)], tidx)
            pltpu.sync_copy(et_hbm.at[pl.ds(base, CHUNK)], etidx)
            ch = pltpu.async_copy(x_hbm.at[hidx], bufh, sem)
            ct = pltpu.async_copy(x_hbm.at[tidx], buft, sem)
            cr = pltpu.async_copy(rel_hbm.at[etidx], bufr, sem)
            ch.wait()
            ct.wait()
            cr.wait()

            for g in range(CHUNK // LANES):
                scores = jnp.zeros((LANES,), jnp.float32)
                for j in range(LANES):
                    e = g * LANES + j
                    acc = None
                    for kk in range(VPR):
                        sl = pl.ds(kk * LANES, LANES)
                        v = bufh[e, sl] + bufr[e, sl] - buft[e, sl]
                        a = jnp.abs(v)
                        acc = a if acc is None else acc + a
                    tot = jnp.sum(acc)
                    scores = jnp.where(lane == j, tot, scores)
                outbuf[pl.ds(g * LANES, LANES)] = scores

            pltpu.sync_copy(outbuf, out_hbm.at[pl.ds(base, CHUNK)])
            return ()

        lax.fori_loop(0, N_ITER, body, (), unroll=False)

    return k


_kernel_call = _make_kernel()


@jax.jit
def kernel(x, edge_index, edge_type, rel_embedding):
    h = edge_index[0]
    t = edge_index[1]
    return _kernel_call(x, h, t, edge_type, rel_embedding)


# speedup vs baseline: 3.8033x; 1.7681x over previous
import functools

import jax
import jax.numpy as jnp
from jax import lax
from jax.experimental import pallas as pl
from jax.experimental.pallas import tpu as pltpu
from jax.experimental.pallas import tpu_sc as plsc

N_NODES = 10000
N_EDGES = 320000
D = 128
NUM_REL = 237

_INFO = plsc.get_sparse_core_info()
NC = _INFO.num_cores
NS = _INFO.num_subcores
NW = NC * NS
LANES = 16
VPR = D // LANES

E_PER_W = N_EDGES // NW
CHUNK = 80
N_ITER = E_PER_W // CHUNK
N_PAIR = (N_ITER - 1) // 2
GROUPS = CHUNK // LANES


def _make_kernel():
    mesh = plsc.VectorSubcoreMesh(core_axis_name="c", subcore_axis_name="s")

    @functools.partial(
        pl.kernel,
        out_type=jax.ShapeDtypeStruct((N_EDGES,), jnp.float32),
        mesh=mesh,
        compiler_params=pltpu.CompilerParams(needs_layout_passes=False),
        scratch_types=[
            pltpu.VMEM((E_PER_W,), jnp.int32),
            pltpu.VMEM((E_PER_W,), jnp.int32),
            pltpu.VMEM((E_PER_W,), jnp.int32),
            pltpu.VMEM((CHUNK, D), jnp.float32),
            pltpu.VMEM((CHUNK, D), jnp.float32),
            pltpu.VMEM((CHUNK, D), jnp.float32),
            pltpu.VMEM((CHUNK, D), jnp.float32),
            pltpu.VMEM((CHUNK, D), jnp.float32),
            pltpu.VMEM((CHUNK, D), jnp.float32),
            pltpu.VMEM((E_PER_W,), jnp.float32),
            pltpu.SemaphoreType.DMA,
            pltpu.SemaphoreType.DMA,
        ],
    )
    def k(x_hbm, h_hbm, t_hbm, et_hbm, rel_hbm, out_hbm,
          hidx, tidx, etidx, bh0, bh1, bt0, bt1, br0, br1, outbuf,
          sem0, sem1):
        wid = lax.axis_index("s") * NC + lax.axis_index("c")
        wbase = wid * E_PER_W

        lane = lax.iota(jnp.int32, 16)
        bufs = ((bh0, bt0, br0, sem0), (bh1, bt1, br1, sem1))

        pltpu.sync_copy(h_hbm.at[pl.ds(wbase, E_PER_W)], hidx)
        pltpu.sync_copy(t_hbm.at[pl.ds(wbase, E_PER_W)], tidx)
        pltpu.sync_copy(et_hbm.at[pl.ds(wbase, E_PER_W)], etidx)

        def fire(i, slot):
            bh, bt, br, sem = bufs[slot]
            sl = pl.ds(i * CHUNK, CHUNK)
            ch = pltpu.async_copy(x_hbm.at[hidx.at[sl]], bh, sem)
            ct = pltpu.async_copy(x_hbm.at[tidx.at[sl]], bt, sem)
            cr = pltpu.async_copy(rel_hbm.at[etidx.at[sl]], br, sem)
            return ch, ct, cr

        def drain(slot):
            bh, bt, br, sem = bufs[slot]
            for b in (bh, bt, br):
                pltpu.make_async_copy(
                    x_hbm.at[hidx.at[pl.ds(0, CHUNK)]], b, sem
                ).wait()

        def compute(i, slot):
            bh, bt, br, _ = bufs[slot]

            def grp(g, _):
                ebase = g * LANES
                scores = jnp.zeros((LANES,), jnp.float32)
                for j in range(LANES):
                    e = ebase + j
                    acc = None
                    for kk in range(VPR):
                        sl = pl.ds(kk * LANES, LANES)
                        v = bh[e, sl] + br[e, sl] - bt[e, sl]
                        a = jnp.abs(v)
                        acc = a if acc is None else acc + a
                    tot = jnp.sum(acc)
                    scores = jnp.where(lane == j, tot, scores)
                outbuf[pl.ds(i * CHUNK + g * LANES, LANES)] = scores
                return ()

            lax.fori_loop(0, GROUPS, grp, (), unroll=False)

        fire(0, 0)

        def pair(p, _):
            i0 = 2 * p
            drain(0)
            fire(i0 + 1, 1)
            compute(i0, 0)
            drain(1)
            fire(i0 + 2, 0)
            compute(i0 + 1, 1)
            return ()

        lax.fori_loop(0, N_PAIR, pair, (), unroll=False)

        drain(0)
        compute(N_ITER - 1, 0)

        pltpu.sync_copy(outbuf, out_hbm.at[pl.ds(wbase, E_PER_W)])

    return k


_kernel_call = _make_kernel()


@jax.jit
def kernel(x, edge_index, edge_type, rel_embedding):
    h = edge_index[0]
    t = edge_index[1]
    return _kernel_call(x, h, t, edge_type, rel_embedding)
